# Initial kernel scaffold; baseline (speedup 1.0000x reference)
import jax
import jax.numpy as jnp
from jax.experimental import pallas as pl


def kernel(input, P, sample, W, b):
    raise NotImplementedError("write your pallas kernel here")


# speedup vs baseline: 3.3604x; 3.3604x over previous
import functools

import jax
import jax.numpy as jnp
from jax.experimental import pallas as pl
from jax.experimental.pallas import tpu as pltpu


def _fused_kernel(xin_ref, w_ref, bias_ref, p_ref, mask_ref, pos_ref,
                  out_ref, xt_ref, *, B, F, n):
    i = pl.program_id(0)

    @pl.when(i == 0)
    def _linear():
        wt = w_ref[...].T
        for bb in range(B):
            sl = slice(bb * F, (bb + 1) * F)
            y = jnp.dot(xin_ref[:, sl], wt,
                        preferred_element_type=jnp.float32)
            xt_ref[:, sl] = y + bias_ref[...]

    p = p_ref[...]
    blkN = p.shape[0]
    col = jax.lax.broadcasted_iota(jnp.int32, (blkN, n), 1)
    onehot = (col == pos_ref[...]).astype(jnp.float32)
    p_eff = jnp.where(mask_ref[...] > 0.5, onehot, p)
    out_ref[...] = jnp.dot(p_eff, xt_ref[...],
                           preferred_element_type=jnp.float32)


def kernel(input, P, sample, W, b):
    Bz, n, F = input.shape
    N = P.shape[0]
    BF = Bz * F
    blkN = 256

    xin = input.transpose(1, 0, 2).reshape(n, BF)
    mask = jnp.zeros((N, 1), jnp.float32).at[sample, 0].set(1.0)
    pos = jnp.zeros((N, 1), jnp.int32).at[sample, 0].set(
        jnp.arange(n, dtype=jnp.int32))

    y2 = pl.pallas_call(
        functools.partial(_fused_kernel, B=Bz, F=F, n=n),
        grid=(N // blkN,),
        in_specs=[
            pl.BlockSpec((n, BF), lambda i: (0, 0)),
            pl.BlockSpec((F, F), lambda i: (0, 0)),
            pl.BlockSpec((1, F), lambda i: (0, 0)),
            pl.BlockSpec((blkN, n), lambda i: (i, 0)),
            pl.BlockSpec((blkN, 1), lambda i: (i, 0)),
            pl.BlockSpec((blkN, 1), lambda i: (i, 0)),
        ],
        out_specs=pl.BlockSpec((blkN, BF), lambda i: (i, 0)),
        out_shape=jax.ShapeDtypeStruct((N, BF), jnp.float32),
        scratch_shapes=[pltpu.VMEM((n, BF), jnp.float32)],
    )(xin, W, b.reshape(1, F), P, mask, pos)

    return y2.reshape(N, Bz, F).transpose(1, 0, 2)


# speedup vs baseline: 3.3802x; 1.0059x over previous
import functools

import jax
import jax.numpy as jnp
from jax.experimental import pallas as pl
from jax.experimental.pallas import tpu as pltpu


def _fused_kernel(xin_ref, w_ref, bias_ref, p_ref, mask_ref, pos_ref,
                  out_ref, xt_ref, *, B, F, n):
    i = pl.program_id(0)

    @pl.when(i == 0)
    def _linear():
        wt = w_ref[...].T
        for bb in range(B):
            sl = slice(bb * F, (bb + 1) * F)
            y = jnp.dot(xin_ref[:, sl], wt,
                        preferred_element_type=jnp.float32)
            xt_ref[:, sl] = (y + bias_ref[...]).astype(jnp.bfloat16)

    p = p_ref[...].astype(jnp.bfloat16)
    blkN = p.shape[0]
    col = jax.lax.broadcasted_iota(jnp.int32, (blkN, n), 1)
    onehot = (col == pos_ref[...]).astype(jnp.bfloat16)
    p_eff = jnp.where(mask_ref[...] > 0.5, onehot, p)
    out_ref[...] = jnp.dot(p_eff, xt_ref[...],
                           preferred_element_type=jnp.float32)


def kernel(input, P, sample, W, b):
    Bz, n, F = input.shape
    N = P.shape[0]
    BF = Bz * F
    blkN = 256

    xin = input.transpose(1, 0, 2).reshape(n, BF)
    mask = jnp.zeros((N, 1), jnp.float32).at[sample, 0].set(1.0)
    pos = jnp.zeros((N, 1), jnp.int32).at[sample, 0].set(
        jnp.arange(n, dtype=jnp.int32))

    y2 = pl.pallas_call(
        functools.partial(_fused_kernel, B=Bz, F=F, n=n),
        grid=(N // blkN,),
        in_specs=[
            pl.BlockSpec((n, BF), lambda i: (0, 0)),
            pl.BlockSpec((F, F), lambda i: (0, 0)),
            pl.BlockSpec((1, F), lambda i: (0, 0)),
            pl.BlockSpec((blkN, n), lambda i: (i, 0)),
            pl.BlockSpec((blkN, 1), lambda i: (i, 0)),
            pl.BlockSpec((blkN, 1), lambda i: (i, 0)),
        ],
        out_specs=pl.BlockSpec((blkN, BF), lambda i: (i, 0)),
        out_shape=jax.ShapeDtypeStruct((N, BF), jnp.float32),
        scratch_shapes=[pltpu.VMEM((n, BF), jnp.bfloat16)],
    )(xin, W, b.reshape(1, F), P, mask, pos)

    return y2.reshape(N, Bz, F).transpose(1, 0, 2)
